# Initial kernel scaffold; baseline (speedup 1.0000x reference)
import jax
import jax.numpy as jnp
from jax.experimental import pallas as pl


def kernel(X, Q, ranks):
    raise NotImplementedError("write your pallas kernel here")


# speedup vs baseline: 5.7473x; 5.7473x over previous
import jax
import jax.numpy as jnp
from jax import lax
from jax.experimental import pallas as pl
from jax.experimental.pallas import tpu as pltpu
from jax.experimental.pallas import tpu_sc as plsc

_M = 30
_MP = 32
_K = 20
_BETA = 0.15
_NEG = -3.0e38


def _sc_gather(X, idx_flat):
    ni = idx_flat.shape[0]
    d = X.shape[1]
    ncores, nsub = 2, 16
    nw = ncores * nsub
    chunk = 128
    b_per_w = ni // nw
    nchunk = b_per_w // chunk
    mesh = plsc.VectorSubcoreMesh(core_axis_name="c", subcore_axis_name="s")
    idx2 = idx_flat.reshape(ni // chunk, chunk)

    @pl.kernel(out_type=jax.ShapeDtypeStruct((ni, d), X.dtype), mesh=mesh,
               scratch_types=[pltpu.VMEM((nchunk, chunk), jnp.int32),
                              pltpu.VMEM((2, chunk, d), X.dtype),
                              pltpu.SemaphoreType.DMA])
    def gk(x_hbm, i_hbm, o_hbm, idx_v, rows_v, sem):
        wid = lax.axis_index("s") * ncores + lax.axis_index("c")
        base = wid * b_per_w
        pltpu.sync_copy(i_hbm.at[pl.ds(wid * nchunk, nchunk)], idx_v)
        pltpu.async_copy(x_hbm.at[idx_v.at[0]], rows_v.at[0], sem)
        for j in range(nchunk):
            pltpu.make_async_copy(x_hbm.at[idx_v.at[j]], rows_v.at[j % 2],
                                  sem).wait()
            if j + 1 < nchunk:
                pltpu.async_copy(x_hbm.at[idx_v.at[j + 1]],
                                 rows_v.at[(j + 1) % 2], sem)
            pltpu.sync_copy(rows_v.at[j % 2],
                            o_hbm.at[pl.ds(base + j * chunk, chunk)])

    return gk(X, idx2)


def _rank_desc_last(v, ndim_shape, i_axis, j_axis):
    ii = lax.broadcasted_iota(jnp.int32, ndim_shape, i_axis)
    jj = lax.broadcasted_iota(jnp.int32, ndim_shape, j_axis)
    vi = jnp.expand_dims(v, j_axis)
    vj = jnp.expand_dims(v, i_axis)
    above = (vi > vj) | ((vi == vj) & (ii < jj))
    return jnp.sum(above.astype(jnp.int32), axis=i_axis)


def _tc_body(xg2_ref, q_ref, idx_ref, final_ref, scores_ref, pre_ref,
             xdba_ref):
    Xg2 = xg2_ref[...]
    B = Xg2.shape[0]
    par = (idx_ref[...] & 1)[:, :, None]
    Xg = jnp.where(par == 0, Xg2[:, :, :64], Xg2[:, :, 64:])
    S = lax.dot_general(Xg, Xg, (((2,), (2,)), ((0,), (0,))),
                        preferred_element_type=jnp.float32)
    j3 = lax.broadcasted_iota(jnp.int32, (B, _MP, _MP), 2)
    V = jnp.where(j3 < _M, S, _NEG)
    R = _rank_desc_last(V, (B, _MP, _MP, _MP), 2, 3)
    W = jnp.where(R == 0, jnp.float32(1.0),
                  jnp.where(R < _K, _BETA * V, jnp.float32(0.0)))
    denom = jnp.sum(W, axis=2)
    num = W[:, :, 0, None] * Xg[:, None, 0, :]
    for j in range(1, _MP):
        num = num + W[:, :, j, None] * Xg[:, None, j, :]
    xdba = num / denom[:, :, None]
    xdba_ref[...] = xdba

    q = q_ref[...]
    sc = jnp.sum(xdba * q[:, None, :], axis=2)
    scores_ref[...] = sc

    m2 = lax.broadcasted_iota(jnp.int32, (B, _MP), 1)
    scm = jnp.where(m2 < _M, sc, _NEG)
    R2 = _rank_desc_last(scm, (B, _MP, _MP), 1, 2)
    p_iota = lax.broadcasted_iota(jnp.int32, (B, _MP, _MP), 1)
    m_iota = lax.broadcasted_iota(jnp.int32, (B, _MP, _MP), 2)
    O = (R2[:, None, :] == p_iota).astype(jnp.int32)
    pre_ref[...] = jnp.sum(O * m_iota, axis=2)
    final_ref[...] = jnp.sum(O * idx_ref[...][:, None, :], axis=2)


def kernel(X, Q, ranks):
    nq = Q.shape[0]
    d = X.shape[1]
    idx = jnp.transpose(ranks[:_M, :]).astype(jnp.int32)
    idx_p = jnp.pad(idx, ((0, 0), (0, _MP - _M)))

    Xp = X.reshape(X.shape[0] // 2, 2 * d)
    Xg2 = _sc_gather(Xp, (idx_p >> 1).reshape(-1)).reshape(nq, _MP, 2 * d)

    B = 8
    nb = nq // B
    final_p, scores_p, pre_p, xdba_p = pl.pallas_call(
        _tc_body,
        grid=(nb,),
        in_specs=[
            pl.BlockSpec((B, _MP, 2 * d), lambda i: (i, 0, 0)),
            pl.BlockSpec((B, d), lambda i: (i, 0)),
            pl.BlockSpec((B, _MP), lambda i: (i, 0)),
        ],
        out_specs=[
            pl.BlockSpec((B, _MP), lambda i: (i, 0)),
            pl.BlockSpec((B, _MP), lambda i: (i, 0)),
            pl.BlockSpec((B, _MP), lambda i: (i, 0)),
            pl.BlockSpec((B, _MP, d), lambda i: (i, 0, 0)),
        ],
        out_shape=[
            jax.ShapeDtypeStruct((nq, _MP), jnp.int32),
            jax.ShapeDtypeStruct((nq, _MP), jnp.float32),
            jax.ShapeDtypeStruct((nq, _MP), jnp.int32),
            jax.ShapeDtypeStruct((nq, _MP, d), jnp.float32),
        ],
        compiler_params=pltpu.CompilerParams(
            dimension_semantics=("parallel",)),
    )(Xg2, Q, idx_p)

    return (final_p[:, :_M], scores_p[:, :_M], pre_p[:, :_M],
            xdba_p[:, :_M, :])


# speedup vs baseline: 12.5114x; 2.1769x over previous
import jax
import jax.numpy as jnp
from jax import lax
from jax.experimental import pallas as pl
from jax.experimental.pallas import tpu as pltpu
from jax.experimental.pallas import tpu_sc as plsc

_M = 30
_MP = 32
_K = 20
_BETA = 0.15
_NEG = -3.0e38


def _sc_gather(X, idx_flat):
    ni = idx_flat.shape[0]
    d = X.shape[1]
    ncores, nsub = 2, 16
    nw = ncores * nsub
    chunk = 128
    b_per_w = ni // nw
    nchunk = b_per_w // chunk
    mesh = plsc.VectorSubcoreMesh(core_axis_name="c", subcore_axis_name="s")
    idx2 = idx_flat.reshape(ni // chunk, chunk)

    @pl.kernel(out_type=jax.ShapeDtypeStruct((ni, d), X.dtype), mesh=mesh,
               scratch_types=[pltpu.VMEM((nchunk, chunk), jnp.int32),
                              pltpu.VMEM((2, chunk, d), X.dtype),
                              pltpu.SemaphoreType.DMA])
    def gk(x_hbm, i_hbm, o_hbm, idx_v, rows_v, sem):
        wid = lax.axis_index("s") * ncores + lax.axis_index("c")
        base = wid * b_per_w
        pltpu.sync_copy(i_hbm.at[pl.ds(wid * nchunk, nchunk)], idx_v)
        pltpu.async_copy(x_hbm.at[idx_v.at[0]], rows_v.at[0], sem)
        for j in range(nchunk):
            pltpu.make_async_copy(x_hbm.at[idx_v.at[j]], rows_v.at[j % 2],
                                  sem).wait()
            if j + 1 < nchunk:
                pltpu.async_copy(x_hbm.at[idx_v.at[j + 1]],
                                 rows_v.at[(j + 1) % 2], sem)
            pltpu.sync_copy(rows_v.at[j % 2],
                            o_hbm.at[pl.ds(base + j * chunk, chunk)])

    return gk(X, idx2)


def _rank_desc_last(v, ndim_shape, i_axis, j_axis):
    ii = lax.broadcasted_iota(jnp.int32, ndim_shape, i_axis)
    jj = lax.broadcasted_iota(jnp.int32, ndim_shape, j_axis)
    vi = jnp.expand_dims(v, j_axis)
    vj = jnp.expand_dims(v, i_axis)
    above = (vi > vj) | ((vi == vj) & (ii < jj))
    return jnp.sum(above.astype(jnp.int32), axis=i_axis)


def _tc_body(xg2_ref, q_ref, idx_ref, final_ref, scores_ref, pre_ref,
             xdba_ref):
    Xg2 = xg2_ref[...]
    B = Xg2.shape[0]
    par = (idx_ref[...] & 1)[:, :, None]
    Xg = jnp.where(par == 0, Xg2[:, :, :64], Xg2[:, :, 64:])
    S = lax.dot_general(Xg, Xg, (((2,), (2,)), ((0,), (0,))),
                        preferred_element_type=jnp.float32)
    j3 = lax.broadcasted_iota(jnp.int32, (B, _MP, _MP), 2)
    V = jnp.where(j3 < _M, S, _NEG)
    R = _rank_desc_last(V, (B, _MP, _MP, _MP), 2, 3)
    W = jnp.where(R == 0, jnp.float32(1.0),
                  jnp.where(R < _K, _BETA * V, jnp.float32(0.0)))
    denom = jnp.sum(W, axis=2)
    num = lax.dot_general(W, Xg, (((2,), (1,)), ((0,), (0,))),
                          precision=lax.Precision.HIGHEST,
                          preferred_element_type=jnp.float32)
    xdba = num / denom[:, :, None]
    xdba_ref[...] = xdba

    q = q_ref[...]
    sc = jnp.sum(xdba * q[:, None, :], axis=2)
    scores_ref[...] = sc

    m2 = lax.broadcasted_iota(jnp.int32, (B, _MP), 1)
    scm = jnp.where(m2 < _M, sc, _NEG)
    R2 = _rank_desc_last(scm, (B, _MP, _MP), 1, 2)
    p_iota = lax.broadcasted_iota(jnp.int32, (B, _MP, _MP), 1)
    m_iota = lax.broadcasted_iota(jnp.int32, (B, _MP, _MP), 2)
    O = (R2[:, None, :] == p_iota).astype(jnp.int32)
    pre_ref[...] = jnp.sum(O * m_iota, axis=2)
    final_ref[...] = jnp.sum(O * idx_ref[...][:, None, :], axis=2)


def kernel(X, Q, ranks):
    nq = Q.shape[0]
    d = X.shape[1]
    idx = jnp.transpose(ranks[:_M, :]).astype(jnp.int32)
    idx_p = jnp.pad(idx, ((0, 0), (0, _MP - _M)))

    Xp = X.reshape(X.shape[0] // 2, 2 * d)
    Xg2 = _sc_gather(Xp, (idx_p >> 1).reshape(-1)).reshape(nq, _MP, 2 * d)

    B = 32
    nb = nq // B
    final_p, scores_p, pre_p, xdba_p = pl.pallas_call(
        _tc_body,
        grid=(nb,),
        in_specs=[
            pl.BlockSpec((B, _MP, 2 * d), lambda i: (i, 0, 0)),
            pl.BlockSpec((B, d), lambda i: (i, 0)),
            pl.BlockSpec((B, _MP), lambda i: (i, 0)),
        ],
        out_specs=[
            pl.BlockSpec((B, _MP), lambda i: (i, 0)),
            pl.BlockSpec((B, _MP), lambda i: (i, 0)),
            pl.BlockSpec((B, _MP), lambda i: (i, 0)),
            pl.BlockSpec((B, _MP, d), lambda i: (i, 0, 0)),
        ],
        out_shape=[
            jax.ShapeDtypeStruct((nq, _MP), jnp.int32),
            jax.ShapeDtypeStruct((nq, _MP), jnp.float32),
            jax.ShapeDtypeStruct((nq, _MP), jnp.int32),
            jax.ShapeDtypeStruct((nq, _MP, d), jnp.float32),
        ],
        compiler_params=pltpu.CompilerParams(
            dimension_semantics=("parallel",)),
    )(Xg2, Q, idx_p)

    return (final_p[:, :_M], scores_p[:, :_M], pre_p[:, :_M],
            xdba_p[:, :_M, :])
